# baseline (speedup 1.0000x reference)
import functools

import jax
import jax.numpy as jnp
from jax import lax
from jax.experimental import pallas as pl
from jax.experimental.pallas import tpu as pltpu
from jax.experimental.pallas import tpu_sc as plsc

N, D, C, R = 8192, 8, 4, 128
T = C + D * (C + 1)
SBLK = 64
NSBLK = N // SBLK
BLK = 128
TBL_ROWS = 768
TBLK = TBL_ROWS // BLK
NC, NS = 2, 16
NW = NC * NS
GTOT = N * C
B_PER_W = GTOT // NW
CH = 128


def _sumsq(x):
    p = x * x
    acc = p[:, 0:8]
    for j in range(1, 16):
        acc = acc + p[:, 8 * j : 8 * j + 8]
    t4 = acc[:, 0:4] + acc[:, 4:8]
    t2 = t4[:, 0:2] + t4[:, 2:4]
    return t2[:, 0:1] + t2[:, 1:2]


def _score_body(nnm, ce, al, mg, cnm, crm, nrm, hre, he, ne_ref, fidx_ref):
    b = pl.program_id(0)
    al_ = al[...].astype(jnp.bfloat16).astype(jnp.float32)
    p = [al_[:, d : d + 1]
         * mg[:, d, :].astype(jnp.bfloat16).astype(jnp.float32)
         for d in range(D)]
    nei_msg = ((p[0] + p[1]) + (p[2] + p[3])) + ((p[4] + p[5]) + (p[6] + p[7]))
    ne = nei_msg + ce[:, 0, :]
    ne_ref[...] = ne

    cols = []
    for t in range(C):
        x = (cnm[:, 0, t, :] + crm[:, 0, t, :]) - ne
        cols.append(_sumsq(x))
    for d in range(D):
        hred = hre[:, d, :]
        for k in range(C):
            x = (nnm[:, d, k, :] + (nrm[:, d, k, :] + hred)) - ne
            cols.append(_sumsq(x))
        x = (he[:, d, :] + hred) - ne
        cols.append(_sumsq(x))
    s = jnp.sqrt(jnp.concatenate(cols, axis=1))

    iota = lax.broadcasted_iota(jnp.int32, (SBLK, T), 1)
    picks = []
    for _ in range(C):
        m = jnp.max(s, axis=1, keepdims=True)
        idx = jnp.min(jnp.where(s == m, iota, T + 1), axis=1, keepdims=True)
        picks.append(idx)
        s = jnp.where(iota == idx, -1.0, s)
    topk = jnp.concatenate(picks, axis=1)
    rows = lax.broadcasted_iota(jnp.int32, (SBLK, C), 0) + b * SBLK
    fidx_ref[...] = rows * C + topk


def _table_body(cnm, crm, nnm, nrm, hre, he, tn_ref, tr_ref):
    tn_ref[:, 0:C, :] = cnm[:, 0]
    tr_ref[:, 0:C, :] = crm[:, 0]
    for d in range(D):
        base = C + (C + 1) * d
        tn_ref[:, base : base + C, :] = nnm[:, d]
        tn_ref[:, base + C, :] = he[:, d]
        tr_ref[:, base : base + C, :] = nrm[:, d] + hre[:, d : d + 1, :]
        tr_ref[:, base + C, :] = hre[:, d]


def _sc_gather_body(tn_hbm, tr_hbm, idx_hbm, outn_hbm, outr_hbm,
                    idx_v, rown_v, rowr_v, semn, semr):
    wid = lax.axis_index("s") * NC + lax.axis_index("c")
    base = wid * B_PER_W
    for cchunk in range(B_PER_W // CH):
        off = base + cchunk * CH
        pltpu.sync_copy(idx_hbm.at[pl.ds(off, CH)], idx_v)
        cpn = pltpu.async_copy(tn_hbm.at[idx_v], rown_v, semn)
        cpr = pltpu.async_copy(tr_hbm.at[idx_v], rowr_v, semr)
        cpn.wait()
        cpr.wait()
        pltpu.sync_copy(rown_v, outn_hbm.at[pl.ds(off, CH)])
        pltpu.sync_copy(rowr_v, outr_hbm.at[pl.ds(off, CH)])


@jax.jit
def kernel(nei_node_mem, curr_emb, alpha, msg, curr_node_mem, curr_rel_mem,
           nei_rel_mem, head_rel_emb, head_emb):
    al2 = alpha[:, :, 0]

    row4 = lambda b: (b, 0, 0, 0)
    row3 = lambda b: (b, 0, 0)
    row2 = lambda b: (b, 0)
    ne, fidx = pl.pallas_call(
        _score_body,
        grid=(NSBLK,),
        in_specs=[
            pl.BlockSpec((SBLK, D, C, R), row4),
            pl.BlockSpec((SBLK, D, R), row3),
            pl.BlockSpec((SBLK, D), row2),
            pl.BlockSpec((SBLK, D, R), row3),
            pl.BlockSpec((SBLK, 1, C, R), row4),
            pl.BlockSpec((SBLK, 1, C, R), row4),
            pl.BlockSpec((SBLK, D, C, R), row4),
            pl.BlockSpec((SBLK, D, R), row3),
            pl.BlockSpec((SBLK, D, R), row3),
        ],
        out_specs=[
            pl.BlockSpec((SBLK, R), row2),
            pl.BlockSpec((SBLK, C), row2),
        ],
        out_shape=[
            jax.ShapeDtypeStruct((N, R), jnp.float32),
            jax.ShapeDtypeStruct((N, C), jnp.int32),
        ],
    )(nei_node_mem, curr_emb, al2, msg, curr_node_mem, curr_rel_mem,
      nei_rel_mem, head_rel_emb, head_emb)

    tn, tr = pl.pallas_call(
        _table_body,
        grid=(TBLK,),
        in_specs=[
            pl.BlockSpec((BLK, 1, C, R), row4),
            pl.BlockSpec((BLK, 1, C, R), row4),
            pl.BlockSpec((BLK, D, C, R), row4),
            pl.BlockSpec((BLK, D, C, R), row4),
            pl.BlockSpec((BLK, D, R), row3),
            pl.BlockSpec((BLK, D, R), row3),
        ],
        out_specs=[
            pl.BlockSpec((BLK, T, R), row3),
            pl.BlockSpec((BLK, T, R), row3),
        ],
        out_shape=[
            jax.ShapeDtypeStruct((TBL_ROWS, T, R), jnp.float32),
            jax.ShapeDtypeStruct((TBL_ROWS, T, R), jnp.float32),
        ],
    )(curr_node_mem[:TBL_ROWS], curr_rel_mem[:TBL_ROWS],
      nei_node_mem[:TBL_ROWS], nei_rel_mem[:TBL_ROWS],
      head_rel_emb[:TBL_ROWS], head_emb[:TBL_ROWS])

    mesh = plsc.VectorSubcoreMesh(core_axis_name="c", subcore_axis_name="s")
    gather = functools.partial(
        pl.kernel,
        mesh=mesh,
        out_type=[
            jax.ShapeDtypeStruct((GTOT, R), jnp.float32),
            jax.ShapeDtypeStruct((GTOT, R), jnp.float32),
        ],
        scratch_types=[
            pltpu.VMEM((CH,), jnp.int32),
            pltpu.VMEM((CH, R), jnp.float32),
            pltpu.VMEM((CH, R), jnp.float32),
            pltpu.SemaphoreType.DMA,
            pltpu.SemaphoreType.DMA,
        ],
    )(_sc_gather_body)
    outn, outr = gather(tn.reshape(TBL_ROWS * T, R), tr.reshape(TBL_ROWS * T, R),
                        fidx.reshape(GTOT))

    return ne, outn.reshape(N, C, R), outr.reshape(N, C, R)


# speedup vs baseline: 5.5910x; 5.5910x over previous
import functools

import jax
import jax.numpy as jnp
from jax import lax
from jax.experimental import pallas as pl
from jax.experimental.pallas import tpu as pltpu
from jax.experimental.pallas import tpu_sc as plsc

N, D, C, R = 8192, 8, 4, 128
T = C + D * (C + 1)
SBLK = 128
NSBLK = N // SBLK
BLK = 128
TBL_ROWS = 768
TBLK = TBL_ROWS // BLK
NC, NS = 2, 16
NW = NC * NS
GTOT = N * C
B_PER_W = GTOT // NW
CH = 128


def _sumsq(x):
    p = x * x
    acc = p[:, 0:8]
    for j in range(1, 16):
        acc = acc + p[:, 8 * j : 8 * j + 8]
    t4 = acc[:, 0:4] + acc[:, 4:8]
    t2 = t4[:, 0:2] + t4[:, 2:4]
    return t2[:, 0:1] + t2[:, 1:2]


def _score_body(nnm, ce, al, mg, cnm, crm, nrm, hre, he, ne_ref, fx_ref):
    b = pl.program_id(0)
    al_ = al[...].astype(jnp.bfloat16).astype(jnp.float32)
    p = [al_[:, d : d + 1]
         * mg[:, d, :].astype(jnp.bfloat16).astype(jnp.float32)
         for d in range(D)]
    nei_msg = ((p[0] + p[1]) + (p[2] + p[3])) + ((p[4] + p[5]) + (p[6] + p[7]))
    ne = nei_msg + ce[:, 0, :]
    ne_ref[...] = ne

    xs = []
    for t in range(C):
        xs.append((cnm[:, 0, t, :] + crm[:, 0, t, :]) - ne)
    for d in range(D):
        hred = hre[:, d, :]
        for k in range(C):
            xs.append((nnm[:, d, k, :] + (nrm[:, d, k, :] + hred)) - ne)
        xs.append((he[:, d, :] + hred) - ne)
    X = jnp.concatenate(xs, axis=0)
    p2 = X * X
    acc = p2[:, 0:8]
    for j in range(1, 16):
        acc = acc + p2[:, 8 * j : 8 * j + 8]
    t4 = acc[:, 0:4] + acc[:, 4:8]
    t2 = t4[:, 0:2] + t4[:, 2:4]
    ss = t2[:, 0:1] + t2[:, 1:2]
    s = jnp.sqrt(ss).reshape(T, SBLK)

    iota = lax.broadcasted_iota(jnp.int32, (T, SBLK), 0)
    picks = []
    for _ in range(C):
        m = jnp.max(s, axis=0, keepdims=True)
        idx = jnp.min(jnp.where(s == m, iota, T + 1), axis=0, keepdims=True)
        picks.append(idx)
        s = jnp.where(iota == idx, -1.0, s)
    topk = jnp.concatenate(picks, axis=0)
    gi = b * SBLK + lax.broadcasted_iota(jnp.int32, (C, SBLK), 1)
    fx = gi * C + topk
    fx_ref[...] = jnp.concatenate([fx, fx], axis=0)


def _table_body(cnm, crm, nnm, nrm, hre, he, tn_ref, tr_ref):
    tn_ref[:, 0:C, :] = cnm[:, 0]
    tr_ref[:, 0:C, :] = crm[:, 0]
    for d in range(D):
        base = C + (C + 1) * d
        tn_ref[:, base : base + C, :] = nnm[:, d]
        tn_ref[:, base + C, :] = he[:, d]
        tr_ref[:, base : base + C, :] = nrm[:, d] + hre[:, d : d + 1, :]
        tr_ref[:, base + C, :] = hre[:, d]


def _sc_gather_body(tn_hbm, tr_hbm, idx_hbm, outn_hbm, outr_hbm,
                    idx_v, rown_v, rowr_v, semn, semr):
    wid = lax.axis_index("s") * NC + lax.axis_index("c")
    base = wid * B_PER_W
    for cchunk in range(B_PER_W // CH):
        off = base + cchunk * CH
        pltpu.sync_copy(idx_hbm.at[pl.ds(off, CH)], idx_v)
        cpn = pltpu.async_copy(tn_hbm.at[idx_v], rown_v, semn)
        cpr = pltpu.async_copy(tr_hbm.at[idx_v], rowr_v, semr)
        cpn.wait()
        cpr.wait()
        pltpu.sync_copy(rown_v, outn_hbm.at[pl.ds(off, CH)])
        pltpu.sync_copy(rowr_v, outr_hbm.at[pl.ds(off, CH)])


@jax.jit
def kernel(nei_node_mem, curr_emb, alpha, msg, curr_node_mem, curr_rel_mem,
           nei_rel_mem, head_rel_emb, head_emb):
    al2 = alpha[:, :, 0]

    row4 = lambda b: (b, 0, 0, 0)
    row3 = lambda b: (b, 0, 0)
    row2 = lambda b: (b, 0)
    ne, fx8 = pl.pallas_call(
        _score_body,
        grid=(NSBLK,),
        in_specs=[
            pl.BlockSpec((SBLK, D, C, R), row4),
            pl.BlockSpec((SBLK, D, R), row3),
            pl.BlockSpec((SBLK, D), row2),
            pl.BlockSpec((SBLK, D, R), row3),
            pl.BlockSpec((SBLK, 1, C, R), row4),
            pl.BlockSpec((SBLK, 1, C, R), row4),
            pl.BlockSpec((SBLK, D, C, R), row4),
            pl.BlockSpec((SBLK, D, R), row3),
            pl.BlockSpec((SBLK, D, R), row3),
        ],
        out_specs=[
            pl.BlockSpec((SBLK, R), row2),
            pl.BlockSpec((8, SBLK), lambda b: (0, b)),
        ],
        out_shape=[
            jax.ShapeDtypeStruct((N, R), jnp.float32),
            jax.ShapeDtypeStruct((8, N), jnp.int32),
        ],
    )(nei_node_mem, curr_emb, al2, msg, curr_node_mem, curr_rel_mem,
      nei_rel_mem, head_rel_emb, head_emb)

    tn, tr = pl.pallas_call(
        _table_body,
        grid=(TBLK,),
        in_specs=[
            pl.BlockSpec((BLK, 1, C, R), row4),
            pl.BlockSpec((BLK, 1, C, R), row4),
            pl.BlockSpec((BLK, D, C, R), row4),
            pl.BlockSpec((BLK, D, C, R), row4),
            pl.BlockSpec((BLK, D, R), row3),
            pl.BlockSpec((BLK, D, R), row3),
        ],
        out_specs=[
            pl.BlockSpec((BLK, T, R), row3),
            pl.BlockSpec((BLK, T, R), row3),
        ],
        out_shape=[
            jax.ShapeDtypeStruct((TBL_ROWS, T, R), jnp.float32),
            jax.ShapeDtypeStruct((TBL_ROWS, T, R), jnp.float32),
        ],
    )(curr_node_mem[:TBL_ROWS], curr_rel_mem[:TBL_ROWS],
      nei_node_mem[:TBL_ROWS], nei_rel_mem[:TBL_ROWS],
      head_rel_emb[:TBL_ROWS], head_emb[:TBL_ROWS])

    mesh = plsc.VectorSubcoreMesh(core_axis_name="c", subcore_axis_name="s")
    gather = functools.partial(
        pl.kernel,
        mesh=mesh,
        out_type=[
            jax.ShapeDtypeStruct((GTOT, R), jnp.float32),
            jax.ShapeDtypeStruct((GTOT, R), jnp.float32),
        ],
        scratch_types=[
            pltpu.VMEM((CH,), jnp.int32),
            pltpu.VMEM((CH, R), jnp.float32),
            pltpu.VMEM((CH, R), jnp.float32),
            pltpu.SemaphoreType.DMA,
            pltpu.SemaphoreType.DMA,
        ],
    )(_sc_gather_body)
    fidx = fx8[:C].T.reshape(GTOT)
    outn, outr = gather(tn.reshape(TBL_ROWS * T, R), tr.reshape(TBL_ROWS * T, R),
                        fidx)

    return ne, outn.reshape(N, C, R), outr.reshape(N, C, R)
